# Initial kernel scaffold; baseline (speedup 1.0000x reference)
import jax
import jax.numpy as jnp
from jax.experimental import pallas as pl


def kernel(pred, target):
    raise NotImplementedError("write your pallas kernel here")


# speedup vs baseline: 29.4703x; 29.4703x over previous
import functools

import jax
import jax.numpy as jnp
from jax import lax
from jax.experimental import pallas as pl
from jax.experimental.pallas import tpu as pltpu
from jax.experimental.pallas import tpu_sc as plsc

_CW0, _CW1, _CW2 = 0.1, 5.0, 5.0
_CSUM = _CW0 + _CW1 + _CW2
_SMOOTH = 1e-06
_OHEM_THRESH = 0.7
_KEEP = 512
_DICE_W = 0.6
_FOCAL_W = 0.4

_B, _C, _H, _W = 16, 3, 512, 512
_N = _B * _H * _W

_NC, _NS, _L = 2, 16, 16
_NW = _NC * _NS
_EPW = _N // _NW
_CHUNK = 16384
_NCHUNK = _EPW // _CHUNK

_BITS1, _BITS2, _BITS3 = 11, 11, 9
_SH1, _SH2 = _BITS2 + _BITS3, _BITS3


def _fused_body(pred_ref, tgt_ref, focal_ref, acc_ref):
    b = pl.program_id(0)
    x0 = pred_ref[0, 0]
    x1 = pred_ref[0, 1]
    x2 = pred_ref[0, 2]
    t = tgt_ref[0]
    m = jnp.maximum(jnp.maximum(x0, x1), x2)
    e0 = jnp.exp(x0 - m)
    e1 = jnp.exp(x1 - m)
    e2 = jnp.exp(x2 - m)
    s = e0 + e1 + e2
    inv = 1.0 / s
    p0 = e0 * inv
    p1 = e1 * inv
    p2 = e2 * inv
    t0 = t == 0
    t1 = t == 1
    xt = jnp.where(t0, x0, jnp.where(t1, x1, x2))
    ce = jnp.log(s) + m - xt
    pt = jnp.where(t0, p0, jnp.where(t1, p1, p2))
    w = jnp.where(t0, _CW0, _CW1)
    omp = 1.0 - pt
    focal = w * omp * omp * ce
    focal_ref[0] = focal

    zero = jnp.float32(0.0)
    hard = focal >= _OHEM_THRESH
    vals = [
        jnp.sum(p0), jnp.sum(p1), jnp.sum(p2),
        jnp.sum(jnp.where(t0, p0, zero)),
        jnp.sum(jnp.where(t1, p1, zero)),
        jnp.sum(jnp.where(t == 2, p2, zero)),
        jnp.sum(t0.astype(jnp.float32)),
        jnp.sum(t1.astype(jnp.float32)),
        jnp.sum((t == 2).astype(jnp.float32)),
        jnp.sum(hard.astype(jnp.float32)),
        jnp.sum(jnp.where(hard, focal, zero)),
    ]

    @pl.when(b == 0)
    def _init():
        for i in range(16):
            acc_ref[i] = zero

    for i, v in enumerate(vals):
        acc_ref[i] += v


_fused = pl.pallas_call(
    _fused_body,
    grid=(_B,),
    in_specs=[
        pl.BlockSpec((1, _C, _H, _W), lambda b: (b, 0, 0, 0)),
        pl.BlockSpec((1, _H, _W), lambda b: (b, 0, 0)),
    ],
    out_specs=[
        pl.BlockSpec((1, _H, _W), lambda b: (b, 0, 0)),
        pl.BlockSpec(memory_space=pltpu.SMEM),
    ],
    out_shape=[
        jax.ShapeDtypeStruct((_B, _H, _W), jnp.float32),
        jax.ShapeDtypeStruct((16,), jnp.float32),
    ],
)


def _make_sc_hist(nbits, shift, use_prefix, prefix_shift):
    nbuck = 1 << nbits
    hsize = nbuck * _L
    bmask = jnp.uint32(nbuck - 1)

    def body(*refs):
        if use_prefix:
            focal_hbm, pref_hbm, cnt_hbm, sum_hbm, buf, pref_v, cnt_v, sum_v = refs
        else:
            focal_hbm, cnt_hbm, sum_hbm, buf, cnt_v, sum_v = refs
        wid = lax.axis_index("s") * _NC + lax.axis_index("c")
        base = wid * _EPW

        zf = jnp.zeros((_L,), jnp.float32)

        def zero_body(j, carry):
            cnt_v[pl.ds(j * _L, _L)] = zf
            sum_v[pl.ds(j * _L, _L)] = zf
            return carry

        lax.fori_loop(0, nbuck, zero_body, 0)

        if use_prefix:
            pltpu.sync_copy(pref_hbm, pref_v)
            pvec = lax.bitcast_convert_type(pref_v[...], jnp.uint32)

        lane = lax.iota(jnp.int32, _L)
        ones = jnp.ones((_L,), jnp.float32)

        def chunk_body(ci, carry):
            pltpu.sync_copy(focal_hbm.at[pl.ds(base + ci * _CHUNK, _CHUNK)], buf)

            def vec_body(i, c2):
                v = buf[pl.ds(i * _L, _L)]
                u = lax.bitcast_convert_type(v, jnp.uint32)
                bk = (u >> shift) & bmask
                idx = (bk.astype(jnp.int32) << 4) + lane
                if use_prefix:
                    pm = (u >> prefix_shift) == pvec
                    plsc.addupdate_scatter(cnt_v, [idx], ones, mask=pm)
                    plsc.addupdate_scatter(sum_v, [idx], v, mask=pm)
                else:
                    plsc.addupdate_scatter(cnt_v, [idx], ones)
                    plsc.addupdate_scatter(sum_v, [idx], v)
                return c2

            lax.fori_loop(0, _CHUNK // _L, vec_body, 0)
            return carry

        lax.fori_loop(0, _NCHUNK, chunk_body, 0)

        pltpu.sync_copy(cnt_v, cnt_hbm.at[wid])
        pltpu.sync_copy(sum_v, sum_hbm.at[wid])

    mesh = plsc.VectorSubcoreMesh(core_axis_name="c", subcore_axis_name="s",
                                  num_cores=_NC, num_subcores=_NS)
    scratch = [pltpu.VMEM((_CHUNK,), jnp.float32)]
    if use_prefix:
        scratch.append(pltpu.VMEM((_L,), jnp.int32))
    scratch += [
        pltpu.VMEM((hsize,), jnp.float32),
        pltpu.VMEM((hsize,), jnp.float32),
    ]
    return pl.kernel(
        body,
        out_type=(
            jax.ShapeDtypeStruct((_NW, hsize), jnp.float32),
            jax.ShapeDtypeStruct((_NW, hsize), jnp.float32),
        ),
        mesh=mesh,
        scratch_types=scratch,
        compiler_params=pltpu.CompilerParams(needs_layout_passes=False),
    )


@functools.lru_cache(maxsize=1)
def _get_hists():
    return (_make_sc_hist(_BITS1, _SH1, False, 0),
            _make_sc_hist(_BITS2, _SH2, True, _SH1),
            _make_sc_hist(_BITS3, 0, True, _SH2))


def _make_search(nbits, rows):
    nbuck = 1 << nbits

    def body(kth_ref, cnt_ref, sum_ref, b_ref, n_ref, s_ref):
        kth = kth_ref[0].astype(jnp.float32)
        cnt = jnp.sum(cnt_ref[...], axis=0)
        sm = jnp.sum(sum_ref[...], axis=0)
        jj = (lax.broadcasted_iota(jnp.int32, (rows, 128), 0) * 128
              + lax.broadcasted_iota(jnp.int32, (rows, 128), 1))

        def suffix_cnt(m):
            return jnp.sum(jnp.where(jj >= m, cnt, jnp.float32(0.0)))

        def search_body(_, carry):
            lo, hi = carry
            mid = (lo + hi) // 2
            ge = suffix_cnt(mid * _L) >= kth
            return (jnp.where(ge, mid, lo), jnp.where(ge, hi, mid))

        lo, hi = lax.fori_loop(0, nbits, search_body,
                               (jnp.int32(0), jnp.int32(nbuck)))
        bsel = lo
        n_above = suffix_cnt((bsel + 1) * _L)
        s_above = jnp.sum(jnp.where(jj >= (bsel + 1) * _L, sm,
                                    jnp.float32(0.0)))
        b_ref[0] = bsel
        n_ref[0] = n_above.astype(jnp.int32)
        s_ref[0] = s_above

    return pl.pallas_call(
        body,
        in_specs=[
            pl.BlockSpec(memory_space=pltpu.SMEM),
            pl.BlockSpec((_NW, rows, 128), lambda: (0, 0, 0)),
            pl.BlockSpec((_NW, rows, 128), lambda: (0, 0, 0)),
        ],
        out_specs=[
            pl.BlockSpec(memory_space=pltpu.SMEM),
            pl.BlockSpec(memory_space=pltpu.SMEM),
            pl.BlockSpec(memory_space=pltpu.SMEM),
        ],
        out_shape=[
            jax.ShapeDtypeStruct((1,), jnp.int32),
            jax.ShapeDtypeStruct((1,), jnp.int32),
            jax.ShapeDtypeStruct((1,), jnp.float32),
        ],
    )


_search1 = _make_search(_BITS1, (1 << _BITS1) * _L // 128)
_search2 = _make_search(_BITS2, (1 << _BITS2) * _L // 128)


def _final_body(scal_ref, acc_ref, cnt_ref, sum_ref, out_ref):
    rows = (1 << _BITS3) * _L // 128
    b1 = scal_ref[0]
    b2 = scal_ref[1]
    n1 = scal_ref[2]
    n2 = scal_ref[3]
    s1f = lax.bitcast_convert_type(scal_ref[4], jnp.float32)
    s2f = lax.bitcast_convert_type(scal_ref[5], jnp.float32)
    kth = (_KEEP - n1 - n2).astype(jnp.float32)

    cnt = jnp.sum(cnt_ref[...], axis=0)
    sm = jnp.sum(sum_ref[...], axis=0)
    jj = (lax.broadcasted_iota(jnp.int32, (rows, 128), 0) * 128
          + lax.broadcasted_iota(jnp.int32, (rows, 128), 1))

    def suffix_cnt(m):
        return jnp.sum(jnp.where(jj >= m, cnt, jnp.float32(0.0)))

    def search_body(_, carry):
        lo, hi = carry
        mid = (lo + hi) // 2
        ge = suffix_cnt(mid * _L) >= kth
        return (jnp.where(ge, mid, lo), jnp.where(ge, hi, mid))

    b3, _ = lax.fori_loop(0, _BITS3, search_body,
                          (jnp.int32(0), jnp.int32(1 << _BITS3)))
    n3 = suffix_cnt((b3 + 1) * _L)
    s3f = jnp.sum(jnp.where(jj >= (b3 + 1) * _L, sm, jnp.float32(0.0)))
    cnt_eq = suffix_cnt(b3 * _L) - n3

    thr_bits = (b1 << (_BITS2 + _BITS3)) | (b2 << _BITS3) | b3
    thr = lax.bitcast_convert_type(thr_bits, jnp.float32)

    cnt_gt = (n1 + n2).astype(jnp.float32) + n3
    s_gt = s1f + s2f + s3f

    dp0, dp1, dp2 = acc_ref[0], acc_ref[1], acc_ref[2]
    di0, di1, di2 = acc_ref[3], acc_ref[4], acc_ref[5]
    dt0, dt1, dt2 = acc_ref[6], acc_ref[7], acc_ref[8]
    c07, s07 = acc_ref[9], acc_ref[10]

    dice0 = (2.0 * di0 + _SMOOTH) / (dp0 + dt0 + _SMOOTH)
    dice1 = (2.0 * di1 + _SMOOTH) / (dp1 + dt1 + _SMOOTH)
    dice2 = (2.0 * di2 + _SMOOTH) / (dp2 + dt2 + _SMOOTH)
    dice_loss = ((1.0 - dice0) * _CW0 + (1.0 - dice1) * _CW1
                 + (1.0 - dice2) * _CW2) / _CSUM

    cnt_eq_f = cnt_eq
    c_ge = cnt_gt + cnt_eq_f
    s_ge = s_gt + cnt_eq_f * thr
    fl_hi = s_ge / jnp.maximum(c_ge, 1.0)
    fl_fb = (s_gt + (jnp.float32(_KEEP) - cnt_gt) * thr) / jnp.float32(_KEEP)
    fl_lo = jnp.where(c07 > 0.0, s07 / jnp.maximum(c07, 1.0), fl_fb)
    focal_loss = jnp.where(thr >= _OHEM_THRESH, fl_hi, fl_lo)

    out_ref[0] = _DICE_W * dice_loss + _FOCAL_W * focal_loss


_final = pl.pallas_call(
    _final_body,
    in_specs=[
        pl.BlockSpec(memory_space=pltpu.SMEM),
        pl.BlockSpec(memory_space=pltpu.SMEM),
        pl.BlockSpec((_NW, (1 << _BITS3) * _L // 128, 128), lambda: (0, 0, 0)),
        pl.BlockSpec((_NW, (1 << _BITS3) * _L // 128, 128), lambda: (0, 0, 0)),
    ],
    out_specs=pl.BlockSpec(memory_space=pltpu.SMEM),
    out_shape=jax.ShapeDtypeStruct((1,), jnp.float32),
)


def kernel(pred, target):
    target = target.astype(jnp.int32)
    _hist1, _hist2, _hist3 = _get_hists()
    focal, acc = _fused(pred, target)
    focal_flat = focal.reshape(_N)

    r1 = (1 << _BITS1) * _L // 128
    cnt1, sum1 = _hist1(focal_flat)
    k1 = jnp.full((1,), _KEEP, jnp.int32)
    b1, n1, s1 = _search1(k1, cnt1.reshape(_NW, r1, 128),
                          sum1.reshape(_NW, r1, 128))

    pref1 = jnp.broadcast_to(b1, (_L,)).astype(jnp.int32)
    r2 = (1 << _BITS2) * _L // 128
    cnt2, sum2 = _hist2(focal_flat, pref1)
    k2 = _KEEP - n1
    b2, n2, s2 = _search2(k2, cnt2.reshape(_NW, r2, 128),
                          sum2.reshape(_NW, r2, 128))

    pref2 = jnp.broadcast_to((b1 << _BITS2) | b2, (_L,)).astype(jnp.int32)
    r3 = (1 << _BITS3) * _L // 128
    cnt3, sum3 = _hist3(focal_flat, pref2)

    scal = jnp.concatenate([
        b1, b2, n1, n2,
        lax.bitcast_convert_type(s1, jnp.int32),
        lax.bitcast_convert_type(s2, jnp.int32),
    ])
    out = _final(scal, acc, cnt3.reshape(_NW, r3, 128),
                 sum3.reshape(_NW, r3, 128))
    return out[0]


# speedup vs baseline: 33.8075x; 1.1472x over previous
import functools

import jax
import jax.numpy as jnp
from jax import lax
from jax.experimental import pallas as pl
from jax.experimental.pallas import tpu as pltpu
from jax.experimental.pallas import tpu_sc as plsc

_CW0, _CW1, _CW2 = 0.1, 5.0, 5.0
_CSUM = _CW0 + _CW1 + _CW2
_SMOOTH = 1e-06
_OHEM_THRESH = 0.7
_KEEP = 512
_DICE_W = 0.6
_FOCAL_W = 0.4

_B, _C, _H, _W = 16, 3, 512, 512
_N = _B * _H * _W

_NC, _NS, _L = 2, 16, 16
_NW = _NC * _NS
_HHALF = _H // 2
_CROWS = 32
_NCHUNK = _HHALF // _CROWS
_VPC = _CROWS * _W // _L

_BITS1, _BITS2, _BITS3 = 11, 11, 9
_SH1, _SH2 = _BITS2 + _BITS3, _BITS3


def _fused_body(pred_ref, tgt_ref, focal_ref, acc_ref):
    b = pl.program_id(0)
    x0 = pred_ref[0, 0]
    x1 = pred_ref[0, 1]
    x2 = pred_ref[0, 2]
    t = tgt_ref[0]
    m = jnp.maximum(jnp.maximum(x0, x1), x2)
    e0 = jnp.exp(x0 - m)
    e1 = jnp.exp(x1 - m)
    e2 = jnp.exp(x2 - m)
    s = e0 + e1 + e2
    inv = 1.0 / s
    p0 = e0 * inv
    p1 = e1 * inv
    p2 = e2 * inv
    t0 = t == 0
    t1 = t == 1
    xt = jnp.where(t0, x0, jnp.where(t1, x1, x2))
    ce = jnp.log(s) + m - xt
    pt = jnp.where(t0, p0, jnp.where(t1, p1, p2))
    w = jnp.where(t0, _CW0, _CW1)
    omp = 1.0 - pt
    focal = w * omp * omp * ce
    focal_ref[0] = focal[:_HHALF]
    focal_ref[1] = focal[_HHALF:]

    zero = jnp.float32(0.0)
    hard = focal >= _OHEM_THRESH
    vals = [
        jnp.sum(p0), jnp.sum(p1), jnp.sum(p2),
        jnp.sum(jnp.where(t0, p0, zero)),
        jnp.sum(jnp.where(t1, p1, zero)),
        jnp.sum(jnp.where(t == 2, p2, zero)),
        jnp.sum(t0.astype(jnp.float32)),
        jnp.sum(t1.astype(jnp.float32)),
        jnp.sum((t == 2).astype(jnp.float32)),
        jnp.sum(hard.astype(jnp.float32)),
        jnp.sum(jnp.where(hard, focal, zero)),
    ]

    @pl.when(b == 0)
    def _init():
        for i in range(16):
            acc_ref[i] = zero

    for i, v in enumerate(vals):
        acc_ref[i] += v


_fused = pl.pallas_call(
    _fused_body,
    grid=(_B,),
    in_specs=[
        pl.BlockSpec((1, _C, _H, _W), lambda b: (b, 0, 0, 0)),
        pl.BlockSpec((1, _H, _W), lambda b: (b, 0, 0)),
    ],
    out_specs=[
        pl.BlockSpec((2, _HHALF, _W), lambda b: (b, 0, 0)),
        pl.BlockSpec(memory_space=pltpu.SMEM),
    ],
    out_shape=[
        jax.ShapeDtypeStruct((_NW, _HHALF, _W), jnp.float32),
        jax.ShapeDtypeStruct((16,), jnp.float32),
    ],
)


def _make_sc_hist(nbits, shift, use_prefix, prefix_shift):
    nbuck = 1 << nbits
    hsize = nbuck * _L
    bmask = jnp.uint32(nbuck - 1)

    def body(*refs):
        if use_prefix:
            (focal_hbm, pref_hbm, cnt_hbm, sum_hbm,
             buf0, buf1, pref_v, cnt_v, sum_v, s0, s1) = refs
        else:
            (focal_hbm, cnt_hbm, sum_hbm,
             buf0, buf1, cnt_v, sum_v, s0, s1) = refs
        wid = lax.axis_index("s") * _NC + lax.axis_index("c")

        def chunk_ref(ci):
            return focal_hbm.at[wid, pl.ds(ci * _CROWS, _CROWS)]

        pltpu.async_copy(chunk_ref(0), buf0, s0)

        zf = jnp.zeros((_L,), jnp.float32)

        def zero_body(j, carry):
            for k in range(8):
                cnt_v[pl.ds((j * 8 + k) * _L, _L)] = zf
                sum_v[pl.ds((j * 8 + k) * _L, _L)] = zf
            return carry

        lax.fori_loop(0, nbuck // 8, zero_body, 0)

        if use_prefix:
            pltpu.sync_copy(pref_hbm, pref_v)
            pvec = lax.bitcast_convert_type(pref_v[...], jnp.uint32)

        lane = lax.iota(jnp.int32, _L)
        ones = jnp.ones((_L,), jnp.float32)

        def process(buf):
            def grp(i, c2):
                for k in range(8):
                    j = i * 8 + k
                    r = j >> 5
                    col = (j & 31) * _L
                    v = buf[r, pl.ds(col, _L)]
                    u = lax.bitcast_convert_type(v, jnp.uint32)
                    bk = (u >> shift) & bmask
                    idx = (bk.astype(jnp.int32) << 4) + lane
                    if use_prefix:
                        pm = (u >> prefix_shift) == pvec
                        plsc.addupdate_scatter(cnt_v, [idx], ones, mask=pm)
                        plsc.addupdate_scatter(sum_v, [idx], v, mask=pm)
                    else:
                        plsc.addupdate_scatter(cnt_v, [idx], ones)
                        plsc.addupdate_scatter(sum_v, [idx], v)
                return c2

            lax.fori_loop(0, _VPC // 8, grp, 0)

        for ci in range(_NCHUNK):
            buf, sem = (buf0, s0) if ci % 2 == 0 else (buf1, s1)
            pltpu.make_async_copy(chunk_ref(ci), buf, sem).wait()
            if ci + 1 < _NCHUNK:
                nbuf, nsem = (buf0, s0) if (ci + 1) % 2 == 0 else (buf1, s1)
                pltpu.async_copy(chunk_ref(ci + 1), nbuf, nsem)
            process(buf)

        pltpu.sync_copy(cnt_v, cnt_hbm.at[wid])
        pltpu.sync_copy(sum_v, sum_hbm.at[wid])

    mesh = plsc.VectorSubcoreMesh(core_axis_name="c", subcore_axis_name="s",
                                  num_cores=_NC, num_subcores=_NS)
    scratch = [pltpu.VMEM((_CROWS, _W), jnp.float32),
               pltpu.VMEM((_CROWS, _W), jnp.float32)]
    if use_prefix:
        scratch.append(pltpu.VMEM((_L,), jnp.int32))
    scratch += [
        pltpu.VMEM((hsize,), jnp.float32),
        pltpu.VMEM((hsize,), jnp.float32),
        pltpu.SemaphoreType.DMA,
        pltpu.SemaphoreType.DMA,
    ]
    return pl.kernel(
        body,
        out_type=(
            jax.ShapeDtypeStruct((_NW, hsize), jnp.float32),
            jax.ShapeDtypeStruct((_NW, hsize), jnp.float32),
        ),
        mesh=mesh,
        scratch_types=scratch,
        compiler_params=pltpu.CompilerParams(needs_layout_passes=False),
    )


@functools.lru_cache(maxsize=1)
def _get_hists():
    return (_make_sc_hist(_BITS1, _SH1, False, 0),
            _make_sc_hist(_BITS2, _SH2, True, _SH1),
            _make_sc_hist(_BITS3, 0, True, _SH2))


def _make_search(nbits, rows):
    nbuck = 1 << nbits

    def body(kth_ref, cnt_ref, sum_ref, b_ref, n_ref, s_ref):
        kth = kth_ref[0].astype(jnp.float32)
        cnt = jnp.sum(cnt_ref[...], axis=0)
        sm = jnp.sum(sum_ref[...], axis=0)
        jj = (lax.broadcasted_iota(jnp.int32, (rows, 128), 0) * 128
              + lax.broadcasted_iota(jnp.int32, (rows, 128), 1))

        def suffix_cnt(m):
            return jnp.sum(jnp.where(jj >= m, cnt, jnp.float32(0.0)))

        def search_body(_, carry):
            lo, hi = carry
            mid = (lo + hi) // 2
            ge = suffix_cnt(mid * _L) >= kth
            return (jnp.where(ge, mid, lo), jnp.where(ge, hi, mid))

        lo, hi = lax.fori_loop(0, nbits, search_body,
                               (jnp.int32(0), jnp.int32(nbuck)))
        bsel = lo
        n_above = suffix_cnt((bsel + 1) * _L)
        s_above = jnp.sum(jnp.where(jj >= (bsel + 1) * _L, sm,
                                    jnp.float32(0.0)))
        b_ref[0] = bsel
        n_ref[0] = n_above.astype(jnp.int32)
        s_ref[0] = s_above

    return pl.pallas_call(
        body,
        in_specs=[
            pl.BlockSpec(memory_space=pltpu.SMEM),
            pl.BlockSpec((_NW, rows, 128), lambda: (0, 0, 0)),
            pl.BlockSpec((_NW, rows, 128), lambda: (0, 0, 0)),
        ],
        out_specs=[
            pl.BlockSpec(memory_space=pltpu.SMEM),
            pl.BlockSpec(memory_space=pltpu.SMEM),
            pl.BlockSpec(memory_space=pltpu.SMEM),
        ],
        out_shape=[
            jax.ShapeDtypeStruct((1,), jnp.int32),
            jax.ShapeDtypeStruct((1,), jnp.int32),
            jax.ShapeDtypeStruct((1,), jnp.float32),
        ],
    )


_search1 = _make_search(_BITS1, (1 << _BITS1) * _L // 128)
_search2 = _make_search(_BITS2, (1 << _BITS2) * _L // 128)


def _final_body(scal_ref, acc_ref, cnt_ref, sum_ref, out_ref):
    rows = (1 << _BITS3) * _L // 128
    b1 = scal_ref[0]
    b2 = scal_ref[1]
    n1 = scal_ref[2]
    n2 = scal_ref[3]
    s1f = lax.bitcast_convert_type(scal_ref[4], jnp.float32)
    s2f = lax.bitcast_convert_type(scal_ref[5], jnp.float32)
    kth = (_KEEP - n1 - n2).astype(jnp.float32)

    cnt = jnp.sum(cnt_ref[...], axis=0)
    sm = jnp.sum(sum_ref[...], axis=0)
    jj = (lax.broadcasted_iota(jnp.int32, (rows, 128), 0) * 128
          + lax.broadcasted_iota(jnp.int32, (rows, 128), 1))

    def suffix_cnt(m):
        return jnp.sum(jnp.where(jj >= m, cnt, jnp.float32(0.0)))

    def search_body(_, carry):
        lo, hi = carry
        mid = (lo + hi) // 2
        ge = suffix_cnt(mid * _L) >= kth
        return (jnp.where(ge, mid, lo), jnp.where(ge, hi, mid))

    b3, _ = lax.fori_loop(0, _BITS3, search_body,
                          (jnp.int32(0), jnp.int32(1 << _BITS3)))
    n3 = suffix_cnt((b3 + 1) * _L)
    s3f = jnp.sum(jnp.where(jj >= (b3 + 1) * _L, sm, jnp.float32(0.0)))
    cnt_eq = suffix_cnt(b3 * _L) - n3

    thr_bits = (b1 << (_BITS2 + _BITS3)) | (b2 << _BITS3) | b3
    thr = lax.bitcast_convert_type(thr_bits, jnp.float32)

    cnt_gt = (n1 + n2).astype(jnp.float32) + n3
    s_gt = s1f + s2f + s3f

    dp0, dp1, dp2 = acc_ref[0], acc_ref[1], acc_ref[2]
    di0, di1, di2 = acc_ref[3], acc_ref[4], acc_ref[5]
    dt0, dt1, dt2 = acc_ref[6], acc_ref[7], acc_ref[8]
    c07, s07 = acc_ref[9], acc_ref[10]

    dice0 = (2.0 * di0 + _SMOOTH) / (dp0 + dt0 + _SMOOTH)
    dice1 = (2.0 * di1 + _SMOOTH) / (dp1 + dt1 + _SMOOTH)
    dice2 = (2.0 * di2 + _SMOOTH) / (dp2 + dt2 + _SMOOTH)
    dice_loss = ((1.0 - dice0) * _CW0 + (1.0 - dice1) * _CW1
                 + (1.0 - dice2) * _CW2) / _CSUM

    cnt_eq_f = cnt_eq
    c_ge = cnt_gt + cnt_eq_f
    s_ge = s_gt + cnt_eq_f * thr
    fl_hi = s_ge / jnp.maximum(c_ge, 1.0)
    fl_fb = (s_gt + (jnp.float32(_KEEP) - cnt_gt) * thr) / jnp.float32(_KEEP)
    fl_lo = jnp.where(c07 > 0.0, s07 / jnp.maximum(c07, 1.0), fl_fb)
    focal_loss = jnp.where(thr >= _OHEM_THRESH, fl_hi, fl_lo)

    out_ref[0] = _DICE_W * dice_loss + _FOCAL_W * focal_loss


_final = pl.pallas_call(
    _final_body,
    in_specs=[
        pl.BlockSpec(memory_space=pltpu.SMEM),
        pl.BlockSpec(memory_space=pltpu.SMEM),
        pl.BlockSpec((_NW, (1 << _BITS3) * _L // 128, 128), lambda: (0, 0, 0)),
        pl.BlockSpec((_NW, (1 << _BITS3) * _L // 128, 128), lambda: (0, 0, 0)),
    ],
    out_specs=pl.BlockSpec(memory_space=pltpu.SMEM),
    out_shape=jax.ShapeDtypeStruct((1,), jnp.float32),
)


def kernel(pred, target):
    target = target.astype(jnp.int32)
    _hist1, _hist2, _hist3 = _get_hists()
    focal, acc = _fused(pred, target)

    r1 = (1 << _BITS1) * _L // 128
    cnt1, sum1 = _hist1(focal)
    k1 = jnp.full((1,), _KEEP, jnp.int32)
    b1, n1, s1 = _search1(k1, cnt1.reshape(_NW, r1, 128),
                          sum1.reshape(_NW, r1, 128))

    pref1 = jnp.broadcast_to(b1, (_L,)).astype(jnp.int32)
    r2 = (1 << _BITS2) * _L // 128
    cnt2, sum2 = _hist2(focal, pref1)
    k2 = _KEEP - n1
    b2, n2, s2 = _search2(k2, cnt2.reshape(_NW, r2, 128),
                          sum2.reshape(_NW, r2, 128))

    pref2 = jnp.broadcast_to((b1 << _BITS2) | b2, (_L,)).astype(jnp.int32)
    r3 = (1 << _BITS3) * _L // 128
    cnt3, sum3 = _hist3(focal, pref2)

    scal = jnp.concatenate([
        b1, b2, n1, n2,
        lax.bitcast_convert_type(s1, jnp.int32),
        lax.bitcast_convert_type(s2, jnp.int32),
    ])
    out = _final(scal, acc, cnt3.reshape(_NW, r3, 128),
                 sum3.reshape(_NW, r3, 128))
    return out[0]


# speedup vs baseline: 34.0853x; 1.0082x over previous
import functools

import jax
import jax.numpy as jnp
from jax import lax
from jax.experimental import pallas as pl
from jax.experimental.pallas import tpu as pltpu
from jax.experimental.pallas import tpu_sc as plsc

_CW0, _CW1, _CW2 = 0.1, 5.0, 5.0
_CSUM = _CW0 + _CW1 + _CW2
_SMOOTH = 1e-06
_OHEM_THRESH = 0.7
_KEEP = 512
_DICE_W = 0.6
_FOCAL_W = 0.4

_B, _C, _H, _W = 16, 3, 512, 512
_N = _B * _H * _W

_NC, _NS, _L = 2, 16, 16
_NW = _NC * _NS
_HHALF = _H // 2
_CROWS = 32
_NCHUNK = _HHALF // _CROWS
_VPC = _CROWS * _W // _L

_BITS1, _BITS2, _BITS3 = 11, 11, 9
_SH1, _SH2 = _BITS2 + _BITS3, _BITS3


def _fused_body(pred_ref, tgt_ref, focal_ref, acc_ref):
    b = pl.program_id(0)
    x0 = pred_ref[0, 0]
    x1 = pred_ref[0, 1]
    x2 = pred_ref[0, 2]
    t = tgt_ref[0]
    m = jnp.maximum(jnp.maximum(x0, x1), x2)
    e0 = jnp.exp(x0 - m)
    e1 = jnp.exp(x1 - m)
    e2 = jnp.exp(x2 - m)
    s = e0 + e1 + e2
    inv = 1.0 / s
    p0 = e0 * inv
    p1 = e1 * inv
    p2 = e2 * inv
    t0 = t == 0
    t1 = t == 1
    xt = jnp.where(t0, x0, jnp.where(t1, x1, x2))
    ce = jnp.log(s) + m - xt
    pt = jnp.where(t0, p0, jnp.where(t1, p1, p2))
    w = jnp.where(t0, _CW0, _CW1)
    omp = 1.0 - pt
    focal = w * omp * omp * ce
    focal_ref[0] = focal[:_HHALF]
    focal_ref[1] = focal[_HHALF:]

    zero = jnp.float32(0.0)
    hard = focal >= _OHEM_THRESH
    vals = [
        jnp.sum(p0), jnp.sum(p1), jnp.sum(p2),
        jnp.sum(jnp.where(t0, p0, zero)),
        jnp.sum(jnp.where(t1, p1, zero)),
        jnp.sum(jnp.where(t == 2, p2, zero)),
        jnp.sum(t0.astype(jnp.float32)),
        jnp.sum(t1.astype(jnp.float32)),
        jnp.sum((t == 2).astype(jnp.float32)),
        jnp.sum(hard.astype(jnp.float32)),
        jnp.sum(jnp.where(hard, focal, zero)),
    ]

    @pl.when(b == 0)
    def _init():
        for i in range(16):
            acc_ref[i] = zero

    for i, v in enumerate(vals):
        acc_ref[i] += v


_fused = pl.pallas_call(
    _fused_body,
    grid=(_B,),
    in_specs=[
        pl.BlockSpec((1, _C, _H, _W), lambda b: (b, 0, 0, 0)),
        pl.BlockSpec((1, _H, _W), lambda b: (b, 0, 0)),
    ],
    out_specs=[
        pl.BlockSpec((2, _HHALF, _W), lambda b: (b, 0, 0)),
        pl.BlockSpec(memory_space=pltpu.SMEM),
    ],
    out_shape=[
        jax.ShapeDtypeStruct((_NW, _HHALF, _W), jnp.float32),
        jax.ShapeDtypeStruct((16,), jnp.float32),
    ],
)


def _make_sc_hist(nbits, shift, use_prefix, prefix_shift):
    nbuck = 1 << nbits
    hsize = nbuck * _L
    bmask = jnp.uint32(nbuck - 1)

    def body(*refs):
        if use_prefix:
            (focal_hbm, pref_hbm, cnt_hbm, sum_hbm,
             buf0, buf1, pref_v, cnt_v, sum_v, s0, s1) = refs
        else:
            (focal_hbm, cnt_hbm, sum_hbm,
             buf0, buf1, cnt_v, sum_v, s0, s1) = refs
        wid = lax.axis_index("s") * _NC + lax.axis_index("c")

        def chunk_ref(ci):
            return focal_hbm.at[wid, pl.ds(ci * _CROWS, _CROWS)]

        pltpu.async_copy(chunk_ref(0), buf0, s0)

        zf = jnp.zeros((_L,), jnp.float32)

        def zero_body(j, carry):
            for k in range(8):
                cnt_v[pl.ds((j * 8 + k) * _L, _L)] = zf
                sum_v[pl.ds((j * 8 + k) * _L, _L)] = zf
            return carry

        lax.fori_loop(0, nbuck // 8, zero_body, 0)

        if use_prefix:
            pltpu.sync_copy(pref_hbm, pref_v)
            pvec = lax.bitcast_convert_type(pref_v[...], jnp.uint32)

        lane = lax.iota(jnp.int32, _L)
        ones = jnp.ones((_L,), jnp.float32)

        def process(buf):
            def row_body(r, c2):
                for k in range(_W // _L):
                    v = buf[r, pl.ds(k * _L, _L)]
                    u = lax.bitcast_convert_type(v, jnp.uint32)
                    bk = (u >> shift) & bmask
                    idx = (bk.astype(jnp.int32) << 4) + lane
                    if use_prefix:
                        pm = (u >> prefix_shift) == pvec
                        plsc.addupdate_scatter(cnt_v, [idx], ones, mask=pm)
                        plsc.addupdate_scatter(sum_v, [idx], v, mask=pm)
                    else:
                        plsc.addupdate_scatter(cnt_v, [idx], ones)
                        plsc.addupdate_scatter(sum_v, [idx], v)
                return c2

            lax.fori_loop(0, _CROWS, row_body, 0)

        for ci in range(_NCHUNK):
            buf, sem = (buf0, s0) if ci % 2 == 0 else (buf1, s1)
            pltpu.make_async_copy(chunk_ref(ci), buf, sem).wait()
            if ci + 1 < _NCHUNK:
                nbuf, nsem = (buf0, s0) if (ci + 1) % 2 == 0 else (buf1, s1)
                pltpu.async_copy(chunk_ref(ci + 1), nbuf, nsem)
            process(buf)

        pltpu.sync_copy(cnt_v, cnt_hbm.at[wid])
        pltpu.sync_copy(sum_v, sum_hbm.at[wid])

    mesh = plsc.VectorSubcoreMesh(core_axis_name="c", subcore_axis_name="s",
                                  num_cores=_NC, num_subcores=_NS)
    scratch = [pltpu.VMEM((_CROWS, _W), jnp.float32),
               pltpu.VMEM((_CROWS, _W), jnp.float32)]
    if use_prefix:
        scratch.append(pltpu.VMEM((_L,), jnp.int32))
    scratch += [
        pltpu.VMEM((hsize,), jnp.float32),
        pltpu.VMEM((hsize,), jnp.float32),
        pltpu.SemaphoreType.DMA,
        pltpu.SemaphoreType.DMA,
    ]
    return pl.kernel(
        body,
        out_type=(
            jax.ShapeDtypeStruct((_NW, hsize), jnp.float32),
            jax.ShapeDtypeStruct((_NW, hsize), jnp.float32),
        ),
        mesh=mesh,
        scratch_types=scratch,
        compiler_params=pltpu.CompilerParams(needs_layout_passes=False),
    )


@functools.lru_cache(maxsize=1)
def _get_hists():
    return (_make_sc_hist(_BITS1, _SH1, False, 0),
            _make_sc_hist(_BITS2, _SH2, True, _SH1),
            _make_sc_hist(_BITS3, 0, True, _SH2))


def _make_search(nbits, rows):
    nbuck = 1 << nbits

    def body(kth_ref, cnt_ref, sum_ref, b_ref, n_ref, s_ref):
        kth = kth_ref[0].astype(jnp.float32)
        cnt = jnp.sum(cnt_ref[...], axis=0)
        sm = jnp.sum(sum_ref[...], axis=0)
        jj = (lax.broadcasted_iota(jnp.int32, (rows, 128), 0) * 128
              + lax.broadcasted_iota(jnp.int32, (rows, 128), 1))

        def suffix_cnt(m):
            return jnp.sum(jnp.where(jj >= m, cnt, jnp.float32(0.0)))

        def search_body(_, carry):
            lo, hi = carry
            mid = (lo + hi) // 2
            ge = suffix_cnt(mid * _L) >= kth
            return (jnp.where(ge, mid, lo), jnp.where(ge, hi, mid))

        lo, hi = lax.fori_loop(0, nbits, search_body,
                               (jnp.int32(0), jnp.int32(nbuck)))
        bsel = lo
        n_above = suffix_cnt((bsel + 1) * _L)
        s_above = jnp.sum(jnp.where(jj >= (bsel + 1) * _L, sm,
                                    jnp.float32(0.0)))
        b_ref[0] = bsel
        n_ref[0] = n_above.astype(jnp.int32)
        s_ref[0] = s_above

    return pl.pallas_call(
        body,
        in_specs=[
            pl.BlockSpec(memory_space=pltpu.SMEM),
            pl.BlockSpec((_NW, rows, 128), lambda: (0, 0, 0)),
            pl.BlockSpec((_NW, rows, 128), lambda: (0, 0, 0)),
        ],
        out_specs=[
            pl.BlockSpec(memory_space=pltpu.SMEM),
            pl.BlockSpec(memory_space=pltpu.SMEM),
            pl.BlockSpec(memory_space=pltpu.SMEM),
        ],
        out_shape=[
            jax.ShapeDtypeStruct((1,), jnp.int32),
            jax.ShapeDtypeStruct((1,), jnp.int32),
            jax.ShapeDtypeStruct((1,), jnp.float32),
        ],
    )


_search1 = _make_search(_BITS1, (1 << _BITS1) * _L // 128)
_search2 = _make_search(_BITS2, (1 << _BITS2) * _L // 128)


def _final_body(scal_ref, acc_ref, cnt_ref, sum_ref, out_ref):
    rows = (1 << _BITS3) * _L // 128
    b1 = scal_ref[0]
    b2 = scal_ref[1]
    n1 = scal_ref[2]
    n2 = scal_ref[3]
    s1f = lax.bitcast_convert_type(scal_ref[4], jnp.float32)
    s2f = lax.bitcast_convert_type(scal_ref[5], jnp.float32)
    kth = (_KEEP - n1 - n2).astype(jnp.float32)

    cnt = jnp.sum(cnt_ref[...], axis=0)
    sm = jnp.sum(sum_ref[...], axis=0)
    jj = (lax.broadcasted_iota(jnp.int32, (rows, 128), 0) * 128
          + lax.broadcasted_iota(jnp.int32, (rows, 128), 1))

    def suffix_cnt(m):
        return jnp.sum(jnp.where(jj >= m, cnt, jnp.float32(0.0)))

    def search_body(_, carry):
        lo, hi = carry
        mid = (lo + hi) // 2
        ge = suffix_cnt(mid * _L) >= kth
        return (jnp.where(ge, mid, lo), jnp.where(ge, hi, mid))

    b3, _ = lax.fori_loop(0, _BITS3, search_body,
                          (jnp.int32(0), jnp.int32(1 << _BITS3)))
    n3 = suffix_cnt((b3 + 1) * _L)
    s3f = jnp.sum(jnp.where(jj >= (b3 + 1) * _L, sm, jnp.float32(0.0)))
    cnt_eq = suffix_cnt(b3 * _L) - n3

    thr_bits = (b1 << (_BITS2 + _BITS3)) | (b2 << _BITS3) | b3
    thr = lax.bitcast_convert_type(thr_bits, jnp.float32)

    cnt_gt = (n1 + n2).astype(jnp.float32) + n3
    s_gt = s1f + s2f + s3f

    dp0, dp1, dp2 = acc_ref[0], acc_ref[1], acc_ref[2]
    di0, di1, di2 = acc_ref[3], acc_ref[4], acc_ref[5]
    dt0, dt1, dt2 = acc_ref[6], acc_ref[7], acc_ref[8]
    c07, s07 = acc_ref[9], acc_ref[10]

    dice0 = (2.0 * di0 + _SMOOTH) / (dp0 + dt0 + _SMOOTH)
    dice1 = (2.0 * di1 + _SMOOTH) / (dp1 + dt1 + _SMOOTH)
    dice2 = (2.0 * di2 + _SMOOTH) / (dp2 + dt2 + _SMOOTH)
    dice_loss = ((1.0 - dice0) * _CW0 + (1.0 - dice1) * _CW1
                 + (1.0 - dice2) * _CW2) / _CSUM

    cnt_eq_f = cnt_eq
    c_ge = cnt_gt + cnt_eq_f
    s_ge = s_gt + cnt_eq_f * thr
    fl_hi = s_ge / jnp.maximum(c_ge, 1.0)
    fl_fb = (s_gt + (jnp.float32(_KEEP) - cnt_gt) * thr) / jnp.float32(_KEEP)
    fl_lo = jnp.where(c07 > 0.0, s07 / jnp.maximum(c07, 1.0), fl_fb)
    focal_loss = jnp.where(thr >= _OHEM_THRESH, fl_hi, fl_lo)

    out_ref[0] = _DICE_W * dice_loss + _FOCAL_W * focal_loss


_final = pl.pallas_call(
    _final_body,
    in_specs=[
        pl.BlockSpec(memory_space=pltpu.SMEM),
        pl.BlockSpec(memory_space=pltpu.SMEM),
        pl.BlockSpec((_NW, (1 << _BITS3) * _L // 128, 128), lambda: (0, 0, 0)),
        pl.BlockSpec((_NW, (1 << _BITS3) * _L // 128, 128), lambda: (0, 0, 0)),
    ],
    out_specs=pl.BlockSpec(memory_space=pltpu.SMEM),
    out_shape=jax.ShapeDtypeStruct((1,), jnp.float32),
)


def kernel(pred, target):
    target = target.astype(jnp.int32)
    _hist1, _hist2, _hist3 = _get_hists()
    focal, acc = _fused(pred, target)

    r1 = (1 << _BITS1) * _L // 128
    cnt1, sum1 = _hist1(focal)
    k1 = jnp.full((1,), _KEEP, jnp.int32)
    b1, n1, s1 = _search1(k1, cnt1.reshape(_NW, r1, 128),
                          sum1.reshape(_NW, r1, 128))

    pref1 = jnp.broadcast_to(b1, (_L,)).astype(jnp.int32)
    r2 = (1 << _BITS2) * _L // 128
    cnt2, sum2 = _hist2(focal, pref1)
    k2 = _KEEP - n1
    b2, n2, s2 = _search2(k2, cnt2.reshape(_NW, r2, 128),
                          sum2.reshape(_NW, r2, 128))

    pref2 = jnp.broadcast_to((b1 << _BITS2) | b2, (_L,)).astype(jnp.int32)
    r3 = (1 << _BITS3) * _L // 128
    cnt3, sum3 = _hist3(focal, pref2)

    scal = jnp.concatenate([
        b1, b2, n1, n2,
        lax.bitcast_convert_type(s1, jnp.int32),
        lax.bitcast_convert_type(s2, jnp.int32),
    ])
    out = _final(scal, acc, cnt3.reshape(_NW, r3, 128),
                 sum3.reshape(_NW, r3, 128))
    return out[0]
